# Initial kernel scaffold; baseline (speedup 1.0000x reference)
import jax
import jax.numpy as jnp
from jax.experimental import pallas as pl


def kernel(means, sample_points, gt):
    raise NotImplementedError("write your pallas kernel here")


# speedup vs baseline: 5.8971x; 5.8971x over previous
import functools

import jax
import jax.numpy as jnp
from jax.experimental import pallas as pl
from jax.experimental.pallas import tpu as pltpu

_K = 200
_BIG = 1e30


def _knn_chamfer_kernel(means_ref, sp_ref, gtT_ref, cd2_ref, cd1_ref,
                        pen_ref, acc_ref, *, B, N, S, M):
    b = pl.program_id(0)

    @pl.when(b == 0)
    def _init():
        for i in range(4):
            acc_ref[i] = 0.0

    D = jnp.zeros((N, M), jnp.float32)
    for c in range(3):
        mcol = means_ref[0, :, c:c + 1]
        grow = gtT_ref[0, c:c + 1, :]
        diff = mcol - grow
        D = D + diff * diff

    iota = jax.lax.broadcasted_iota(jnp.int32, (N, M), 1)

    rowmin = jnp.min(D, axis=1, keepdims=True)
    e1_sum = jnp.sum(jnp.sqrt(jnp.maximum(rowmin, 1e-9)))
    colmin = jnp.min(D, axis=0, keepdims=True)
    e2_sum = jnp.sum(jnp.sqrt(jnp.maximum(colmin, 1e-9)))

    idx0 = jnp.min(jnp.where(D == rowmin, iota, M), axis=1, keepdims=True)

    bits = jax.lax.bitcast_convert_type(D, jnp.int32)
    lo = jnp.full((N, 1), -1, jnp.int32)
    hi = jnp.full((N, 1), 0x7F7FFFFF, jnp.int32)

    def vbody(_, carry):
        lo, hi = carry
        mid = lo + (hi - lo) // 2
        cnt = jnp.sum((bits <= mid).astype(jnp.int32), axis=1, keepdims=True)
        ge = cnt >= (_K + 1)
        return jnp.where(ge, lo, mid), jnp.where(ge, mid, hi)

    lo, hi = jax.lax.fori_loop(0, 31, vbody, (lo, hi))
    thr = hi

    cnt_lt = jnp.sum((bits < thr).astype(jnp.int32), axis=1, keepdims=True)
    need = (_K + 1) - cnt_lt
    eq = bits == thr
    lo2 = jnp.full((N, 1), -1, jnp.int32)
    hi2 = jnp.full((N, 1), M - 1, jnp.int32)

    def ibody(_, carry):
        lo, hi = carry
        mid = lo + (hi - lo) // 2
        cnt = jnp.sum((eq & (iota <= mid)).astype(jnp.int32), axis=1,
                      keepdims=True)
        ge = cnt >= need
        return jnp.where(ge, lo, mid), jnp.where(ge, mid, hi)

    lo2, hi2 = jax.lax.fori_loop(0, 13, ibody, (lo2, hi2))

    sel = (bits < thr) | (eq & (iota <= hi2))
    sel = sel & (iota != idx0)
    pen_ref[...] = jnp.where(sel, 0.0, _BIG)

    def nbody(n, carry):
        s1, s2 = carry
        spn = sp_ref[0, pl.ds(n * S, S), :]
        pen_row = pen_ref[pl.ds(n, 1), :]
        dd = jnp.zeros((S, M), jnp.float32)
        for c in range(3):
            diff = spn[:, c:c + 1] - gtT_ref[0, c:c + 1, :]
            dd = dd + diff * diff
        d1 = jnp.min(dd + pen_row, axis=1, keepdims=True)
        s1 = s1 + jnp.sum(jnp.sqrt(jnp.maximum(d1, 1e-9)))
        mins = jnp.min(dd, axis=0, keepdims=True)
        d2 = jnp.where(pen_row == 0.0,
                       jnp.sqrt(jnp.maximum(mins, 1e-9)), 0.0)
        s2 = s2 + jnp.sum(d2)
        return s1, s2

    s1, s2 = jax.lax.fori_loop(0, N, nbody, (0.0, 0.0))

    acc_ref[0] = acc_ref[0] + s1
    acc_ref[1] = acc_ref[1] + s2
    acc_ref[2] = acc_ref[2] + e1_sum
    acc_ref[3] = acc_ref[3] + e2_sum

    @pl.when(b == B - 1)
    def _finish():
        S1, S2 = acc_ref[0], acc_ref[1]
        E1, E2 = acc_ref[2], acc_ref[3]
        cd2 = (S1 / (B * S) + S2 / (B * _K)) * 0.5 * 1000.0
        cd1 = (E1 / (B * N) + E2 / (B * M)) * 0.5 * 1000.0
        cd2_ref[...] = jnp.reshape(cd2, (1, 1))
        cd1_ref[...] = jnp.reshape(cd1, (1, 1))


@jax.jit
def kernel(means, sample_points, gt):
    B, N, S, _ = sample_points.shape
    M = gt.shape[1]
    sp = sample_points.reshape(B, N * S, 3)
    gtT = gt.transpose(0, 2, 1)

    body = functools.partial(_knn_chamfer_kernel, B=B, N=N, S=S, M=M)
    cd2, cd1 = pl.pallas_call(
        body,
        grid=(B,),
        in_specs=[
            pl.BlockSpec((1, N, 3), lambda b: (b, 0, 0)),
            pl.BlockSpec((1, N * S, 3), lambda b: (b, 0, 0)),
            pl.BlockSpec((1, 3, M), lambda b: (b, 0, 0)),
        ],
        out_specs=[
            pl.BlockSpec((1, 1), lambda b: (0, 0)),
            pl.BlockSpec((1, 1), lambda b: (0, 0)),
        ],
        out_shape=[
            jax.ShapeDtypeStruct((1, 1), jnp.float32),
            jax.ShapeDtypeStruct((1, 1), jnp.float32),
        ],
        scratch_shapes=[
            pltpu.VMEM((N, M), jnp.float32),
            pltpu.SMEM((4,), jnp.float32),
        ],
        compiler_params=pltpu.CompilerParams(
            dimension_semantics=("arbitrary",)),
    )(means, sp, gtT)

    cd2s = cd2[0, 0]
    cd1s = cd1[0, 0]
    return (cd2s, cd1s, cd2s)


# speedup vs baseline: 10.1947x; 1.7288x over previous
import functools

import jax
import jax.numpy as jnp
from jax.experimental import pallas as pl
from jax.experimental.pallas import tpu as pltpu

_K = 200
_BIG = 1e30


def _knn_chamfer_kernel(means_ref, sp_ref, gtT_ref, cd2_ref, cd1_ref,
                        pen_ref, acc_ref, *, B, N, S, M):
    b = pl.program_id(0)

    @pl.when(b == 0)
    def _init():
        for i in range(4):
            acc_ref[i] = 0.0

    D = jnp.zeros((N, M), jnp.float32)
    for c in range(3):
        mcol = means_ref[0, :, c:c + 1]
        grow = gtT_ref[0, c:c + 1, :]
        diff = mcol - grow
        D = D + diff * diff

    iota = jax.lax.broadcasted_iota(jnp.int32, (N, M), 1)

    rowmin = jnp.min(D, axis=1, keepdims=True)
    e1_sum = jnp.sum(jnp.sqrt(jnp.maximum(rowmin, 1e-9)))
    colmin = jnp.min(D, axis=0, keepdims=True)
    e2_sum = jnp.sum(jnp.sqrt(jnp.maximum(colmin, 1e-9)))

    idx0 = jnp.min(jnp.where(D == rowmin, iota, M), axis=1, keepdims=True)

    bits = jax.lax.bitcast_convert_type(D, jnp.int32)
    lo = jnp.full((N, 1), -1, jnp.int32)
    hi = jnp.full((N, 1), 0x7F7FFFFF, jnp.int32)

    def vbody(_, carry):
        lo, hi = carry
        mid = lo + (hi - lo) // 2
        cnt = jnp.sum((bits <= mid).astype(jnp.int32), axis=1, keepdims=True)
        ge = cnt >= (_K + 1)
        return jnp.where(ge, lo, mid), jnp.where(ge, mid, hi)

    lo, hi = jax.lax.fori_loop(0, 31, vbody, (lo, hi))
    thr = hi

    cnt_lt = jnp.sum((bits < thr).astype(jnp.int32), axis=1, keepdims=True)
    need = (_K + 1) - cnt_lt
    eq = bits == thr
    lo2 = jnp.full((N, 1), -1, jnp.int32)
    hi2 = jnp.full((N, 1), M - 1, jnp.int32)

    def ibody(_, carry):
        lo, hi = carry
        mid = lo + (hi - lo) // 2
        cnt = jnp.sum((eq & (iota <= mid)).astype(jnp.int32), axis=1,
                      keepdims=True)
        ge = cnt >= need
        return jnp.where(ge, lo, mid), jnp.where(ge, mid, hi)

    lo2, hi2 = jax.lax.fori_loop(0, 13, ibody, (lo2, hi2))

    sel = (bits < thr) | (eq & (iota <= hi2))
    sel = sel & (iota != idx0)
    pen_ref[...] = jnp.where(sel, 0.0, _BIG)

    g = gtT_ref[0]
    G4 = jnp.concatenate([-2.0 * g, jnp.sum(g * g, axis=0, keepdims=True)],
                         axis=0)
    GRP = 8
    R = GRP * S

    def nbody(t, carry):
        s1, s2 = carry
        spc = sp_ref[0, pl.ds(t * R, R), :]
        spc4 = jnp.concatenate(
            [spc, jnp.ones((R, 1), jnp.float32)], axis=1)
        s2col = jnp.sum(spc * spc, axis=1, keepdims=True)
        dd = jnp.dot(spc4, G4,
                     preferred_element_type=jnp.float32) + s2col
        penc = pen_ref[pl.ds(t * GRP, GRP), :]
        pen_exp = jnp.broadcast_to(
            penc[:, None, :], (GRP, S, M)).reshape(R, M)
        d1 = jnp.min(dd + pen_exp, axis=1, keepdims=True)
        s1 = s1 + jnp.sum(jnp.sqrt(jnp.maximum(d1, 1e-9)))
        for i in range(GRP):
            mins = jnp.min(dd[i * S:(i + 1) * S], axis=0,
                           keepdims=True)
            s2 = s2 + jnp.sum(
                jnp.where(penc[i:i + 1] == 0.0,
                          jnp.sqrt(jnp.maximum(mins, 1e-9)), 0.0))
        return s1, s2

    s1, s2 = jax.lax.fori_loop(0, N // GRP, nbody, (0.0, 0.0))

    acc_ref[0] = acc_ref[0] + s1
    acc_ref[1] = acc_ref[1] + s2
    acc_ref[2] = acc_ref[2] + e1_sum
    acc_ref[3] = acc_ref[3] + e2_sum

    @pl.when(b == B - 1)
    def _finish():
        S1, S2 = acc_ref[0], acc_ref[1]
        E1, E2 = acc_ref[2], acc_ref[3]
        cd2 = (S1 / (B * S) + S2 / (B * _K)) * 0.5 * 1000.0
        cd1 = (E1 / (B * N) + E2 / (B * M)) * 0.5 * 1000.0
        cd2_ref[...] = jnp.reshape(cd2, (1, 1))
        cd1_ref[...] = jnp.reshape(cd1, (1, 1))


@jax.jit
def kernel(means, sample_points, gt):
    B, N, S, _ = sample_points.shape
    M = gt.shape[1]
    sp = sample_points.reshape(B, N * S, 3)
    gtT = gt.transpose(0, 2, 1)

    body = functools.partial(_knn_chamfer_kernel, B=B, N=N, S=S, M=M)
    cd2, cd1 = pl.pallas_call(
        body,
        grid=(B,),
        in_specs=[
            pl.BlockSpec((1, N, 3), lambda b: (b, 0, 0)),
            pl.BlockSpec((1, N * S, 3), lambda b: (b, 0, 0)),
            pl.BlockSpec((1, 3, M), lambda b: (b, 0, 0)),
        ],
        out_specs=[
            pl.BlockSpec((1, 1), lambda b: (0, 0)),
            pl.BlockSpec((1, 1), lambda b: (0, 0)),
        ],
        out_shape=[
            jax.ShapeDtypeStruct((1, 1), jnp.float32),
            jax.ShapeDtypeStruct((1, 1), jnp.float32),
        ],
        scratch_shapes=[
            pltpu.VMEM((N, M), jnp.float32),
            pltpu.SMEM((4,), jnp.float32),
        ],
        compiler_params=pltpu.CompilerParams(
            dimension_semantics=("arbitrary",)),
    )(means, sp, gtT)

    cd2s = cd2[0, 0]
    cd1s = cd1[0, 0]
    return (cd2s, cd1s, cd2s)


# speedup vs baseline: 11.5822x; 1.1361x over previous
import functools

import jax
import jax.numpy as jnp
from jax.experimental import pallas as pl
from jax.experimental.pallas import tpu as pltpu

_K = 200
_BIG = 1e30


def _knn_chamfer_kernel(means_ref, sp_ref, gtT_ref, cd2_ref, cd1_ref,
                        pen_ref, acc_ref, *, B, N, S, M):
    b = pl.program_id(0)

    @pl.when(b == 0)
    def _init():
        for i in range(4):
            acc_ref[i] = 0.0

    D = jnp.zeros((N, M), jnp.float32)
    for c in range(3):
        mcol = means_ref[0, :, c:c + 1]
        grow = gtT_ref[0, c:c + 1, :]
        diff = mcol - grow
        D = D + diff * diff

    iota = jax.lax.broadcasted_iota(jnp.int32, (N, M), 1)

    rowmin = jnp.min(D, axis=1, keepdims=True)
    e1_sum = jnp.sum(jnp.sqrt(jnp.maximum(rowmin, 1e-9)))
    colmin = jnp.min(D, axis=0, keepdims=True)
    e2_sum = jnp.sum(jnp.sqrt(jnp.maximum(colmin, 1e-9)))

    idx0 = jnp.min(jnp.where(D == rowmin, iota, M), axis=1, keepdims=True)

    bits = jax.lax.bitcast_convert_type(D, jnp.int32)
    lo = jax.lax.bitcast_convert_type(rowmin, jnp.int32) - 1
    hi = jax.lax.bitcast_convert_type(
        jnp.max(D, axis=1, keepdims=True), jnp.int32)

    def vcond(carry):
        lo, hi = carry
        return jnp.max(hi - lo) > 1

    def vbody(carry):
        lo, hi = carry
        mid = lo + (hi - lo) // 2
        cnt = jnp.sum((bits <= mid).astype(jnp.int32), axis=1, keepdims=True)
        ge = cnt >= (_K + 1)
        return jnp.where(ge, lo, mid), jnp.where(ge, mid, hi)

    lo, hi = jax.lax.while_loop(vcond, vbody, (lo, hi))
    thr = hi

    eq = bits == thr
    n_eq = jnp.sum(eq.astype(jnp.int32))

    def no_ties():
        return jnp.full((N, 1), M - 1, jnp.int32)

    def with_ties():
        cnt_lt = jnp.sum((bits < thr).astype(jnp.int32), axis=1,
                         keepdims=True)
        need = (_K + 1) - cnt_lt
        lo2 = jnp.full((N, 1), -1, jnp.int32)
        hi2 = jnp.full((N, 1), M - 1, jnp.int32)

        def ibody(_, carry):
            lo, hi = carry
            mid = lo + (hi - lo) // 2
            cnt = jnp.sum((eq & (iota <= mid)).astype(jnp.int32), axis=1,
                          keepdims=True)
            ge = cnt >= need
            return jnp.where(ge, lo, mid), jnp.where(ge, mid, hi)

        lo2, hi2 = jax.lax.fori_loop(0, 13, ibody, (lo2, hi2))
        return hi2

    istar = jax.lax.cond(n_eq == N, no_ties, with_ties)
    sel = (bits < thr) | (eq & (iota <= istar))
    sel = sel & (iota != idx0)
    pen_ref[...] = jnp.where(sel, 0.0, _BIG)

    g = gtT_ref[0]
    G4 = jnp.concatenate([-2.0 * g, jnp.sum(g * g, axis=0, keepdims=True)],
                         axis=0)
    GRP = 8
    R = GRP * S

    def nbody(t, carry):
        s1, s2 = carry
        spc = sp_ref[0, pl.ds(t * R, R), :]
        spc4 = jnp.concatenate(
            [spc, jnp.ones((R, 1), jnp.float32)], axis=1)
        s2col = jnp.sum(spc * spc, axis=1, keepdims=True)
        dd = jnp.dot(spc4, G4,
                     preferred_element_type=jnp.float32) + s2col
        penc = pen_ref[pl.ds(t * GRP, GRP), :]
        pen_exp = jnp.broadcast_to(
            penc[:, None, :], (GRP, S, M)).reshape(R, M)
        d1 = jnp.min(dd + pen_exp, axis=1, keepdims=True)
        s1 = s1 + jnp.sum(jnp.sqrt(jnp.maximum(d1, 1e-9)))
        for i in range(GRP):
            mins = jnp.min(dd[i * S:(i + 1) * S], axis=0,
                           keepdims=True)
            s2 = s2 + jnp.sum(
                jnp.where(penc[i:i + 1] == 0.0,
                          jnp.sqrt(jnp.maximum(mins, 1e-9)), 0.0))
        return s1, s2

    s1, s2 = jax.lax.fori_loop(0, N // GRP, nbody, (0.0, 0.0))

    acc_ref[0] = acc_ref[0] + s1
    acc_ref[1] = acc_ref[1] + s2
    acc_ref[2] = acc_ref[2] + e1_sum
    acc_ref[3] = acc_ref[3] + e2_sum

    @pl.when(b == B - 1)
    def _finish():
        S1, S2 = acc_ref[0], acc_ref[1]
        E1, E2 = acc_ref[2], acc_ref[3]
        cd2 = (S1 / (B * S) + S2 / (B * _K)) * 0.5 * 1000.0
        cd1 = (E1 / (B * N) + E2 / (B * M)) * 0.5 * 1000.0
        cd2_ref[...] = jnp.reshape(cd2, (1, 1))
        cd1_ref[...] = jnp.reshape(cd1, (1, 1))


@jax.jit
def kernel(means, sample_points, gt):
    B, N, S, _ = sample_points.shape
    M = gt.shape[1]
    sp = sample_points.reshape(B, N * S, 3)
    gtT = gt.transpose(0, 2, 1)

    body = functools.partial(_knn_chamfer_kernel, B=B, N=N, S=S, M=M)
    cd2, cd1 = pl.pallas_call(
        body,
        grid=(B,),
        in_specs=[
            pl.BlockSpec((1, N, 3), lambda b: (b, 0, 0)),
            pl.BlockSpec((1, N * S, 3), lambda b: (b, 0, 0)),
            pl.BlockSpec((1, 3, M), lambda b: (b, 0, 0)),
        ],
        out_specs=[
            pl.BlockSpec((1, 1), lambda b: (0, 0)),
            pl.BlockSpec((1, 1), lambda b: (0, 0)),
        ],
        out_shape=[
            jax.ShapeDtypeStruct((1, 1), jnp.float32),
            jax.ShapeDtypeStruct((1, 1), jnp.float32),
        ],
        scratch_shapes=[
            pltpu.VMEM((N, M), jnp.float32),
            pltpu.SMEM((4,), jnp.float32),
        ],
        compiler_params=pltpu.CompilerParams(
            dimension_semantics=("arbitrary",)),
    )(means, sp, gtT)

    cd2s = cd2[0, 0]
    cd1s = cd1[0, 0]
    return (cd2s, cd1s, cd2s)
